# baseline (speedup 1.0000x reference)
import functools

import jax
import jax.numpy as jnp
from jax import lax
from jax.experimental import pallas as pl
from jax.experimental.pallas import tpu as pltpu
from jax.experimental.pallas import tpu_sc as plsc

B = 4096
U = 4096
I = 8192
D = 64

NC = 2
NS = 16
NW = NC * NS
L = 16
BPW = B // NW
NG = BPW // L

_mesh = plsc.VectorSubcoreMesh(core_axis_name="c", subcore_axis_name="s")


@functools.partial(
    pl.kernel,
    mesh=_mesh,
    out_type=(
        jax.ShapeDtypeStruct((B,), jnp.float32),
        jax.ShapeDtypeStruct((B,), jnp.float32),
    ),
    scratch_types=dict(
        idx_u=pltpu.VMEM((BPW,), jnp.int32),
        idx_i=pltpu.VMEM((BPW,), jnp.int32),
        idx_j=pltpu.VMEM((BPW,), jnp.int32),
        lin_v=pltpu.VMEM((U + I,), jnp.float32),
        u_rows=pltpu.VMEM((BPW, D), jnp.float32),
        ei_rows=pltpu.VMEM((BPW, D), jnp.float32),
        ej_rows=pltpu.VMEM((BPW, D), jnp.float32),
        out_i_v=pltpu.VMEM((BPW,), jnp.float32),
        out_j_v=pltpu.VMEM((BPW,), jnp.float32),
        sem=pltpu.SemaphoreType.DMA,
    ),
    compiler_params=pltpu.CompilerParams(
        needs_layout_passes=False, use_tc_tiling_on_sc=False),
)
def _fm_kernel(
    user_hbm, item_i_hbm, item_j_hbm, lin_hbm, eu_hbm, eit_hbm,
    out_i_hbm, out_j_hbm,
    *, idx_u, idx_i, idx_j, lin_v, u_rows, ei_rows, ej_rows,
    out_i_v, out_j_v, sem,
):
    wid = lax.axis_index("s") * NC + lax.axis_index("c")
    base = wid * BPW

    stage = [
        pltpu.async_copy(user_hbm.at[pl.ds(base, BPW)], idx_u, sem),
        pltpu.async_copy(item_i_hbm.at[pl.ds(base, BPW)], idx_i, sem),
        pltpu.async_copy(item_j_hbm.at[pl.ds(base, BPW)], idx_j, sem),
        pltpu.async_copy(lin_hbm.at[0], lin_v, sem),
    ]
    for cp in stage:
        cp.wait()

    cps = [
        pltpu.async_copy(eu_hbm.at[idx_u], u_rows, sem),
        pltpu.async_copy(eit_hbm.at[idx_i], ei_rows, sem),
        pltpu.async_copy(eit_hbm.at[idx_j], ej_rows, sem),
    ]

    off_u = jnp.full((L,), U, jnp.int32)
    scale = jnp.full((L,), 0.3, jnp.float32)
    for g in range(NG):
        sl = pl.ds(g * L, L)
        lu = plsc.load_gather(lin_v, [idx_u[sl]])
        li = plsc.load_gather(lin_v, [idx_i[sl] + off_u])
        lj = plsc.load_gather(lin_v, [idx_j[sl] + off_u])
        out_i_v[sl] = scale * (lu + li)
        out_j_v[sl] = scale * (lu + lj)

    for cp in cps:
        cp.wait()

    iota = lax.iota(jnp.int32, L)
    for g in range(NG):
        rows = iota + jnp.full((L,), g * L, jnp.int32)
        acc_i = None
        acc_j = None
        for d in range(D):
            dcol = jnp.full((L,), d, jnp.int32)
            ucol = plsc.load_gather(u_rows, [rows, dcol])
            eicol = plsc.load_gather(ei_rows, [rows, dcol])
            ejcol = plsc.load_gather(ej_rows, [rows, dcol])
            if acc_i is None:
                acc_i = ucol * eicol
                acc_j = ucol * ejcol
            else:
                acc_i = acc_i + ucol * eicol
                acc_j = acc_j + ucol * ejcol
        sl = pl.ds(g * L, L)
        out_i_v[sl] = out_i_v[sl] + acc_i
        out_j_v[sl] = out_j_v[sl] + acc_j

    pltpu.sync_copy(out_i_v, out_i_hbm.at[pl.ds(base, BPW)])
    pltpu.sync_copy(out_j_v, out_j_hbm.at[pl.ds(base, BPW)])


def kernel(user, item_i, item_j, linear_w, embed_user_w, embed_item_w):
    user = user.astype(jnp.int32)
    item_i = item_i.astype(jnp.int32)
    item_j = item_j.astype(jnp.int32)
    return _fm_kernel(user, item_i, item_j, linear_w, embed_user_w,
                      embed_item_w)
